# baseline (speedup 1.0000x reference)
import jax
import jax.numpy as jnp
from jax import lax
from jax.experimental import pallas as pl
from jax.experimental.pallas import tpu as pltpu
from jax.experimental.pallas import tpu_sc as plsc

NC, NS, LANES = 2, 16, 16
NW = NC * NS
N0A, N0B = 4096, 4096
B0, B1 = 512, 1024
CW = 64
NSWEEP = N0B // (NW * CW)
RCHUNK = 64
NCHUNK = N0A // RCHUNK
GROUPS = CW // LANES
SSE_BCHUNK = 64
NSSE = B1 // SSE_BCHUNK


def _sc_body(mu0, obs1, idx0, idx1, out_p1, out_b0,
             buf, i1b, i0b, acc1, bin0, stage):
    c = lax.axis_index("c")
    s = lax.axis_index("s")
    w = c * NS + s
    iota = lax.iota(jnp.int32, LANES)
    zv = jnp.zeros((LANES,), jnp.float32)

    def z1(i, _):
        acc1[i, pl.ds(0, LANES)] = zv
        acc1[i, pl.ds(LANES, LANES)] = zv
        acc1[i, pl.ds(2 * LANES, LANES)] = zv
        acc1[i, pl.ds(3 * LANES, LANES)] = zv
        return 0
    lax.fori_loop(0, B1, z1, 0)

    def z0(i, _):
        bin0[i, :] = zv
        return 0
    lax.fori_loop(0, B0, z0, 0)

    loss = zv

    for sweep in range(NSWEEP):
        cbase = sweep * (NW * CW) + w * CW

        def chunk_body(k, _):
            r0 = k * RCHUNK
            pltpu.sync_copy(idx1.at[pl.ds(r0, RCHUNK)], i1b)
            pltpu.sync_copy(idx0.at[pl.ds(r0, RCHUNK)], i0b)
            pltpu.sync_copy(mu0.at[pl.ds(r0, RCHUNK), pl.ds(cbase, CW)], buf)

            def grp(j16, _):
                bv = i1b[pl.ds(j16 * LANES, LANES)]
                b0v = i0b[pl.ds(j16 * LANES, LANES)]
                for j in range(LANES):
                    r = j16 * LANES + j
                    row1 = jnp.full((LANES,), bv[j], jnp.int32)
                    g0 = buf[r, pl.ds(0, LANES)]
                    g1 = buf[r, pl.ds(LANES, LANES)]
                    g2 = buf[r, pl.ds(2 * LANES, LANES)]
                    g3 = buf[r, pl.ds(3 * LANES, LANES)]
                    plsc.addupdate_scatter(acc1, [row1, iota], g0)
                    plsc.addupdate_scatter(acc1, [row1, iota + LANES], g1)
                    plsc.addupdate_scatter(acc1, [row1, iota + 2 * LANES], g2)
                    plsc.addupdate_scatter(acc1, [row1, iota + 3 * LANES], g3)
                    rsv = (g0 + g1) + (g2 + g3)
                    row0 = jnp.full((LANES,), b0v[j], jnp.int32)
                    plsc.addupdate_scatter(bin0, [row0, iota], rsv)
                return 0
            lax.fori_loop(0, RCHUNK // LANES, grp, 0)
            return 0
        lax.fori_loop(0, NCHUNK, chunk_body, 0)

        def sse_chunk(kb, carry):
            b0bin = kb * SSE_BCHUNK
            pltpu.sync_copy(obs1.at[pl.ds(b0bin, SSE_BCHUNK), pl.ds(cbase, CW)], buf)

            def sse_row(rb, acc):
                for g in range(GROUPS):
                    a = acc1[b0bin + rb, pl.ds(g * LANES, LANES)]
                    o = buf[rb, pl.ds(g * LANES, LANES)]
                    d = o - a
                    acc = acc + d * d
                return acc
            return lax.fori_loop(0, SSE_BCHUNK, sse_row, carry)
        loss = lax.fori_loop(0, NSSE, sse_chunk, loss)

        if sweep + 1 < NSWEEP:
            lax.fori_loop(0, B1, z1, 0)

    stage[:] = loss
    pltpu.sync_copy(stage, out_p1.at[w])
    pltpu.sync_copy(bin0, out_b0.at[w])


def _sc_call(mu0, obs1, idx0, idx1):
    mesh = plsc.VectorSubcoreMesh(core_axis_name="c", subcore_axis_name="s",
                                  num_cores=NC, num_subcores=NS)
    return pl.kernel(
        _sc_body,
        out_type=(jax.ShapeDtypeStruct((NW, LANES), jnp.float32),
                  jax.ShapeDtypeStruct((NW, B0, LANES), jnp.float32)),
        mesh=mesh,
        compiler_params=pltpu.CompilerParams(needs_layout_passes=False,
                                             use_tc_tiling_on_sc=False),
        scratch_types=[
            pltpu.VMEM((RCHUNK, CW), jnp.float32),
            pltpu.VMEM((RCHUNK,), jnp.int32),
            pltpu.VMEM((RCHUNK,), jnp.int32),
            pltpu.VMEM((B1, CW), jnp.float32),
            pltpu.VMEM((B0, LANES), jnp.float32),
            pltpu.VMEM((LANES,), jnp.float32),
        ],
    )(mu0, obs1, idx0, idx1)


def _tc_body(mu1_ref, map2_ref, obs0_ref, obs2_ref, p1_ref, b0p_ref, out_ref):
    colsum = jnp.sum(mu1_ref[...], axis=0, keepdims=True)
    proj = lax.dot_general(map2_ref[...], colsum,
                           (((1,), (1,)), ((), ())),
                           preferred_element_type=jnp.float32)
    d2 = obs2_ref[...] - proj[:, 0]
    sse2 = jnp.sum(d2 * d2)
    bin0 = jnp.sum(b0p_ref[...], axis=(0, 2))
    d0 = obs0_ref[...] - bin0
    sse0 = jnp.sum(d0 * d0)
    sse1 = jnp.sum(p1_ref[...])
    total = 0.5 * (sse0 + sse1) + sse2
    out_ref[...] = jnp.reshape(total, (1, 1))


def _tc_call(mu1, mapping2, obs0, obs2, p1, b0p):
    return pl.pallas_call(
        _tc_body,
        out_shape=jax.ShapeDtypeStruct((1, 1), jnp.float32),
    )(mu1, mapping2, obs0, obs2, p1, b0p)


def kernel(mu0, mu1, obs0, obs1, obs2, idx0, idx1, mapping2):
    p1, b0p = _sc_call(mu0, obs1, idx0, idx1)
    loss = _tc_call(mu1, mapping2, obs0, obs2, p1, b0p)
    return loss[0, 0]


# speedup vs baseline: 1.7419x; 1.7419x over previous
import jax
import jax.numpy as jnp
from jax import lax
from jax.experimental import pallas as pl
from jax.experimental.pallas import tpu as pltpu
from jax.experimental.pallas import tpu_sc as plsc

NC, NS, LANES = 2, 16, 16
NW = NC * NS
N0A, N0B = 4096, 4096
B0, B1 = 512, 1024
CW = 64
NSWEEP = N0B // (NW * CW)
RCHUNK = 64
NCHUNK = N0A // RCHUNK
GROUPS = CW // LANES
SSE_BCHUNK = 64
NSSE = B1 // SSE_BCHUNK
NBUF = 4


def _sc_body(mu0, obs1, idx0, idx1, out_p1, out_b0,
             b0, b1, b2, b3, i1all, i0all, acc1, bin0, stage,
             s0, s1, s2, s3):
    c = lax.axis_index("c")
    s = lax.axis_index("s")
    w = c * NS + s
    iota = lax.iota(jnp.int32, LANES)
    zv = jnp.zeros((LANES,), jnp.float32)
    bufs = (b0, b1, b2, b3)
    sems = (s0, s1, s2, s3)

    pltpu.sync_copy(idx1, i1all)
    pltpu.sync_copy(idx0, i0all)

    def z1(i, _):
        for g in range(GROUPS):
            acc1[i, pl.ds(g * LANES, LANES)] = zv
        return 0
    lax.fori_loop(0, B1, z1, 0)

    def z0(i, _):
        bin0[i, :] = zv
        return 0
    lax.fori_loop(0, B0, z0, 0)

    loss = zv

    for sweep in range(NSWEEP):
        cbase = sweep * (NW * CW) + w * CW

        def mu_src(k):
            return mu0.at[pl.ds(k * RCHUNK, RCHUNK), pl.ds(cbase, CW)]

        def scatter_chunk(k, buf):
            def grp(j16, _):
                r0 = k * RCHUNK + j16 * LANES
                bv = i1all[pl.ds(r0, LANES)]
                b0v = i0all[pl.ds(r0, LANES)]
                for j in range(LANES):
                    r = j16 * LANES + j
                    row1 = jnp.full((LANES,), bv[j], jnp.int32)
                    g0 = buf[r, pl.ds(0, LANES)]
                    g1 = buf[r, pl.ds(LANES, LANES)]
                    g2 = buf[r, pl.ds(2 * LANES, LANES)]
                    g3 = buf[r, pl.ds(3 * LANES, LANES)]
                    plsc.addupdate_scatter(acc1, [row1, iota], g0)
                    plsc.addupdate_scatter(acc1, [row1, iota + LANES], g1)
                    plsc.addupdate_scatter(acc1, [row1, iota + 2 * LANES], g2)
                    plsc.addupdate_scatter(acc1, [row1, iota + 3 * LANES], g3)
                    rsv = (g0 + g1) + (g2 + g3)
                    row0 = jnp.full((LANES,), b0v[j], jnp.int32)
                    plsc.addupdate_scatter(bin0, [row0, iota], rsv)
                return 0
            lax.fori_loop(0, RCHUNK // LANES, grp, 0)

        for b in range(NBUF):
            pltpu.async_copy(mu_src(b), bufs[b], sems[b])

        def ring_iter(nb, _):
            for b in range(NBUF):
                k = nb * NBUF + b
                pltpu.make_async_copy(mu_src(k), bufs[b], sems[b]).wait()
                scatter_chunk(k, bufs[b])
                pltpu.async_copy(mu_src(k + NBUF), bufs[b], sems[b])
            return 0
        lax.fori_loop(0, NCHUNK // NBUF - 1, ring_iter, 0)
        for b in range(NBUF):
            k = NCHUNK - NBUF + b
            pltpu.make_async_copy(mu_src(k), bufs[b], sems[b]).wait()
            scatter_chunk(k, bufs[b])

        def obs_src(kb):
            return obs1.at[pl.ds(kb * SSE_BCHUNK, SSE_BCHUNK), pl.ds(cbase, CW)]

        def sse_chunk(kb, buf, carry):
            def sse_row(rb, acc):
                for g in range(GROUPS):
                    a = acc1[kb * SSE_BCHUNK + rb, pl.ds(g * LANES, LANES)]
                    o = buf[rb, pl.ds(g * LANES, LANES)]
                    d = o - a
                    acc = acc + d * d
                return acc
            return lax.fori_loop(0, SSE_BCHUNK, sse_row, carry)

        for b in range(NBUF):
            pltpu.async_copy(obs_src(b), bufs[b], sems[b])

        def sse_ring(nb, carry):
            for b in range(NBUF):
                kb = nb * NBUF + b
                pltpu.make_async_copy(obs_src(kb), bufs[b], sems[b]).wait()
                carry = sse_chunk(kb, bufs[b], carry)
                pltpu.async_copy(obs_src(kb + NBUF), bufs[b], sems[b])
            return carry
        loss = lax.fori_loop(0, NSSE // NBUF - 1, sse_ring, loss)
        for b in range(NBUF):
            kb = NSSE - NBUF + b
            pltpu.make_async_copy(obs_src(kb), bufs[b], sems[b]).wait()
            loss = sse_chunk(kb, bufs[b], loss)

        if sweep + 1 < NSWEEP:
            lax.fori_loop(0, B1, z1, 0)

    stage[:] = loss
    pltpu.sync_copy(stage, out_p1.at[w])
    pltpu.sync_copy(bin0, out_b0.at[w])


def _sc_call(mu0, obs1, idx0, idx1):
    mesh = plsc.VectorSubcoreMesh(core_axis_name="c", subcore_axis_name="s",
                                  num_cores=NC, num_subcores=NS)
    return pl.kernel(
        _sc_body,
        out_type=(jax.ShapeDtypeStruct((NW, LANES), jnp.float32),
                  jax.ShapeDtypeStruct((NW, B0, LANES), jnp.float32)),
        mesh=mesh,
        compiler_params=pltpu.CompilerParams(needs_layout_passes=False,
                                             use_tc_tiling_on_sc=False),
        scratch_types=[
            pltpu.VMEM((RCHUNK, CW), jnp.float32),
            pltpu.VMEM((RCHUNK, CW), jnp.float32),
            pltpu.VMEM((RCHUNK, CW), jnp.float32),
            pltpu.VMEM((RCHUNK, CW), jnp.float32),
            pltpu.VMEM((N0A,), jnp.int32),
            pltpu.VMEM((N0A,), jnp.int32),
            pltpu.VMEM((B1, CW), jnp.float32),
            pltpu.VMEM((B0, LANES), jnp.float32),
            pltpu.VMEM((LANES,), jnp.float32),
            pltpu.SemaphoreType.DMA,
            pltpu.SemaphoreType.DMA,
            pltpu.SemaphoreType.DMA,
            pltpu.SemaphoreType.DMA,
        ],
    )(mu0, obs1, idx0, idx1)


def _tc_body(mu1_ref, map2_ref, obs0_ref, obs2_ref, p1_ref, b0p_ref, out_ref):
    colsum = jnp.sum(mu1_ref[...], axis=0, keepdims=True)
    proj = lax.dot_general(map2_ref[...], colsum,
                           (((1,), (1,)), ((), ())),
                           preferred_element_type=jnp.float32)
    d2 = obs2_ref[...] - proj[:, 0]
    sse2 = jnp.sum(d2 * d2)
    bin0 = jnp.sum(b0p_ref[...], axis=(0, 2))
    d0 = obs0_ref[...] - bin0
    sse0 = jnp.sum(d0 * d0)
    sse1 = jnp.sum(p1_ref[...])
    total = 0.5 * (sse0 + sse1) + sse2
    out_ref[...] = jnp.reshape(total, (1, 1))


def _tc_call(mu1, mapping2, obs0, obs2, p1, b0p):
    return pl.pallas_call(
        _tc_body,
        out_shape=jax.ShapeDtypeStruct((1, 1), jnp.float32),
    )(mu1, mapping2, obs0, obs2, p1, b0p)


def kernel(mu0, mu1, obs0, obs1, obs2, idx0, idx1, mapping2):
    p1, b0p = _sc_call(mu0, obs1, idx0, idx1)
    loss = _tc_call(mu1, mapping2, obs0, obs2, p1, b0p)
    return loss[0, 0]


# speedup vs baseline: 1.9015x; 1.0916x over previous
import jax
import jax.numpy as jnp
from jax import lax
from jax.experimental import pallas as pl
from jax.experimental.pallas import tpu as pltpu
from jax.experimental.pallas import tpu_sc as plsc

NC, NS, LANES = 2, 16, 16
NW = NC * NS
N0A, N0B = 4096, 4096
B0, B1 = 512, 1024
CW = 64
NSWEEP = N0B // (NW * CW)
RCHUNK = 128
NCHUNK = N0A // RCHUNK
GROUPS = CW // LANES
SSE_BCHUNK = 128
NSSE = B1 // SSE_BCHUNK
NBUF = 4


def _sc_body(mu0, obs1, idx0, idx1, out_p1, out_b0,
             b0, b1, b2, b3, i1all, i0all, acc1, bin0, stage,
             s0, s1, s2, s3):
    c = lax.axis_index("c")
    s = lax.axis_index("s")
    w = c * NS + s
    iota = lax.iota(jnp.int32, LANES)
    zv = jnp.zeros((LANES,), jnp.float32)
    bufs = (b0, b1, b2, b3)
    sems = (s0, s1, s2, s3)

    pltpu.sync_copy(idx1, i1all)
    pltpu.sync_copy(idx0, i0all)

    def z1(i, _):
        for g in range(GROUPS):
            acc1[i, pl.ds(g * LANES, LANES)] = zv
        return 0
    lax.fori_loop(0, B1, z1, 0)

    def z0(i, _):
        bin0[i, :] = zv
        return 0
    lax.fori_loop(0, B0, z0, 0)

    loss = zv

    for sweep in range(NSWEEP):
        cbase = sweep * (NW * CW) + w * CW

        def mu_src(k):
            return mu0.at[pl.ds(k * RCHUNK, RCHUNK), pl.ds(cbase, CW)]

        def scatter_chunk(k, buf):
            def grp(j16, _):
                r0 = k * RCHUNK + j16 * LANES
                bv = i1all[pl.ds(r0, LANES)]
                b0v = i0all[pl.ds(r0, LANES)]
                for j in range(LANES):
                    r = j16 * LANES + j
                    row1 = jnp.full((LANES,), bv[j], jnp.int32)
                    g0 = buf[r, pl.ds(0, LANES)]
                    g1 = buf[r, pl.ds(LANES, LANES)]
                    g2 = buf[r, pl.ds(2 * LANES, LANES)]
                    g3 = buf[r, pl.ds(3 * LANES, LANES)]
                    plsc.addupdate_scatter(acc1, [row1, iota], g0)
                    plsc.addupdate_scatter(acc1, [row1, iota + LANES], g1)
                    plsc.addupdate_scatter(acc1, [row1, iota + 2 * LANES], g2)
                    plsc.addupdate_scatter(acc1, [row1, iota + 3 * LANES], g3)
                    rsv = (g0 + g1) + (g2 + g3)
                    row0 = jnp.full((LANES,), b0v[j], jnp.int32)
                    plsc.addupdate_scatter(bin0, [row0, iota], rsv)
                return 0
            lax.fori_loop(0, RCHUNK // LANES, grp, 0)

        for b in range(NBUF):
            pltpu.async_copy(mu_src(b), bufs[b], sems[b])

        def ring_iter(nb, _):
            for b in range(NBUF):
                k = nb * NBUF + b
                pltpu.make_async_copy(mu_src(k), bufs[b], sems[b]).wait()
                scatter_chunk(k, bufs[b])
                pltpu.async_copy(mu_src(k + NBUF), bufs[b], sems[b])
            return 0
        lax.fori_loop(0, NCHUNK // NBUF - 1, ring_iter, 0)
        for b in range(NBUF):
            k = NCHUNK - NBUF + b
            pltpu.make_async_copy(mu_src(k), bufs[b], sems[b]).wait()
            scatter_chunk(k, bufs[b])

        def obs_src(kb):
            return obs1.at[pl.ds(kb * SSE_BCHUNK, SSE_BCHUNK), pl.ds(cbase, CW)]

        def sse_chunk(kb, buf, carry):
            def sse_row(rb, acc):
                for g in range(GROUPS):
                    a = acc1[kb * SSE_BCHUNK + rb, pl.ds(g * LANES, LANES)]
                    o = buf[rb, pl.ds(g * LANES, LANES)]
                    d = o - a
                    acc = acc + d * d
                return acc
            return lax.fori_loop(0, SSE_BCHUNK, sse_row, carry)

        for b in range(NBUF):
            pltpu.async_copy(obs_src(b), bufs[b], sems[b])

        def sse_ring(nb, carry):
            for b in range(NBUF):
                kb = nb * NBUF + b
                pltpu.make_async_copy(obs_src(kb), bufs[b], sems[b]).wait()
                carry = sse_chunk(kb, bufs[b], carry)
                pltpu.async_copy(obs_src(kb + NBUF), bufs[b], sems[b])
            return carry
        loss = lax.fori_loop(0, NSSE // NBUF - 1, sse_ring, loss)
        for b in range(NBUF):
            kb = NSSE - NBUF + b
            pltpu.make_async_copy(obs_src(kb), bufs[b], sems[b]).wait()
            loss = sse_chunk(kb, bufs[b], loss)

        if sweep + 1 < NSWEEP:
            lax.fori_loop(0, B1, z1, 0)

    stage[:] = loss
    pltpu.sync_copy(stage, out_p1.at[w])
    pltpu.sync_copy(bin0, out_b0.at[w])


def _sc_call(mu0, obs1, idx0, idx1):
    mesh = plsc.VectorSubcoreMesh(core_axis_name="c", subcore_axis_name="s",
                                  num_cores=NC, num_subcores=NS)
    return pl.kernel(
        _sc_body,
        out_type=(jax.ShapeDtypeStruct((NW, LANES), jnp.float32),
                  jax.ShapeDtypeStruct((NW, B0, LANES), jnp.float32)),
        mesh=mesh,
        compiler_params=pltpu.CompilerParams(needs_layout_passes=False,
                                             use_tc_tiling_on_sc=False),
        scratch_types=[
            pltpu.VMEM((RCHUNK, CW), jnp.float32),
            pltpu.VMEM((RCHUNK, CW), jnp.float32),
            pltpu.VMEM((RCHUNK, CW), jnp.float32),
            pltpu.VMEM((RCHUNK, CW), jnp.float32),
            pltpu.VMEM((N0A,), jnp.int32),
            pltpu.VMEM((N0A,), jnp.int32),
            pltpu.VMEM((B1, CW), jnp.float32),
            pltpu.VMEM((B0, LANES), jnp.float32),
            pltpu.VMEM((LANES,), jnp.float32),
            pltpu.SemaphoreType.DMA,
            pltpu.SemaphoreType.DMA,
            pltpu.SemaphoreType.DMA,
            pltpu.SemaphoreType.DMA,
        ],
    )(mu0, obs1, idx0, idx1)


def _tc_body(mu1_ref, map2_ref, obs0_ref, obs2_ref, p1_ref, b0p_ref, out_ref):
    colsum = jnp.sum(mu1_ref[...], axis=0, keepdims=True)
    proj = lax.dot_general(map2_ref[...], colsum,
                           (((1,), (1,)), ((), ())),
                           preferred_element_type=jnp.float32)
    d2 = obs2_ref[...] - proj[:, 0]
    sse2 = jnp.sum(d2 * d2)
    bin0 = jnp.sum(b0p_ref[...], axis=(0, 2))
    d0 = obs0_ref[...] - bin0
    sse0 = jnp.sum(d0 * d0)
    sse1 = jnp.sum(p1_ref[...])
    total = 0.5 * (sse0 + sse1) + sse2
    out_ref[...] = jnp.reshape(total, (1, 1))


def _tc_call(mu1, mapping2, obs0, obs2, p1, b0p):
    return pl.pallas_call(
        _tc_body,
        out_shape=jax.ShapeDtypeStruct((1, 1), jnp.float32),
    )(mu1, mapping2, obs0, obs2, p1, b0p)


def kernel(mu0, mu1, obs0, obs1, obs2, idx0, idx1, mapping2):
    p1, b0p = _sc_call(mu0, obs1, idx0, idx1)
    loss = _tc_call(mu1, mapping2, obs0, obs2, p1, b0p)
    return loss[0, 0]
